# baseline (speedup 1.0000x reference)
import functools

import jax
import jax.numpy as jnp
from jax import lax
from jax.experimental import pallas as pl
from jax.experimental.pallas import tpu as pltpu
from jax.experimental.pallas import tpu_sc as plsc

B, S, P, F = 4, 2048, 16, 128
N = B * S
D = P * F
NT = P + 1

_info = plsc.get_sparse_core_info()
NC, NS, L = _info.num_cores, _info.num_subcores, _info.num_lanes
NW = NC * NS
RPW = N // NW
CH = 16


def _sc_body(x_hbm, pad_hbm, tab_hbm, out_hbm, tab_v, pad_v, buf_v):
    wid = lax.axis_index("s") * NC + lax.axis_index("c")
    base = wid * RPW
    pltpu.sync_copy(tab_hbm, tab_v)
    pltpu.sync_copy(pad_hbm.at[pl.ds(base, RPW)], pad_v)

    def chunk_body(g, _):
        row0 = base + g * CH
        pltpu.sync_copy(x_hbm.at[pl.ds(row0, CH)], buf_v)

        def row_body(r, _):
            prow = pad_v[g * CH + r, :]
            nz = plsc.all_reduce_population_count(prow != 0)
            cnt = P - nz[0]

            def col_body(j, _):
                sl = pl.ds(j * L, L)
                buf_v[r, sl] = buf_v[r, sl] + tab_v[cnt, sl]
                return 0

            lax.fori_loop(0, D // L, col_body, 0)
            return 0

        lax.fori_loop(0, CH, row_body, 0)
        pltpu.sync_copy(buf_v, out_hbm.at[pl.ds(row0, CH)])
        return 0

    lax.fori_loop(0, RPW // CH, chunk_body, 0)


@jax.jit
def _sc_call(x, pad, table):
    mesh = plsc.VectorSubcoreMesh(core_axis_name="c", subcore_axis_name="s")
    kern = functools.partial(
        pl.kernel,
        mesh=mesh,
        out_type=jax.ShapeDtypeStruct((N, D), jnp.float32),
        scratch_types=[
            pltpu.VMEM((NT, D), jnp.float32),
            pltpu.VMEM((RPW, P), jnp.int32),
            pltpu.VMEM((CH, D), jnp.float32),
        ],
        compiler_params=pltpu.CompilerParams(needs_layout_passes=False),
    )(_sc_body)
    return kern(x, pad, table)


def kernel(inputs, paddings, table):
    x = inputs.reshape(N, D)
    pad = paddings.reshape(N, P)
    out = _sc_call(x, pad, table)
    return out.reshape(B, S, D)
